# Initial kernel scaffold; baseline (speedup 1.0000x reference)
import jax
import jax.numpy as jnp
from jax.experimental import pallas as pl


def kernel(image, w1, w2):
    raise NotImplementedError("write your pallas kernel here")


# speedup vs baseline: 5.1197x; 5.1197x over previous
import math

import jax
import jax.numpy as jnp
import numpy as np
from jax.experimental import pallas as pl
from jax.experimental.pallas import tpu as pltpu

_IN_SHAPE = (224, 224)
_RADIUS = 0.005
_KSHAPE = (3, 4)
_C = 96
_H, _W = _IN_SHAPE
_WP = 256
_RB = 16
_NBLK = _H // _RB
_NPIX = _H * _W


def _disco_psi(in_shape, radius_cutoff, kernel_shape):
    H, W = in_shape
    nr, nphi = kernel_shape
    n_basis = (nr - 1) * nphi + 1
    rh = max(1, int(math.floor(radius_cutoff * (H - 1))))
    rw = max(1, int(math.floor(radius_cutoff * (W - 1))))
    dy = np.arange(-rh, rh + 1) / float(H - 1)
    dx = np.arange(-rw, rw + 1) / float(W - 1)
    Y, X = np.meshgrid(dy, dx, indexing="ij")
    r = np.sqrt(X ** 2 + Y ** 2) / radius_cutoff
    phi = np.arctan2(Y, X) % (2.0 * np.pi)
    dr = 1.0 / (nr - 1)
    dphi = 2.0 * np.pi / nphi
    psi = np.zeros((n_basis, 2 * rh + 1, 2 * rw + 1), dtype=np.float32)
    support = (r <= 1.0).astype(np.float32)
    psi[0] = np.maximum(0.0, 1.0 - r / dr) * support
    for k in range(1, n_basis):
        ir = (k - 1) // nphi + 1
        iphi = (k - 1) % nphi
        rv = np.maximum(0.0, 1.0 - np.abs(r - ir * dr) / dr) * support
        dp = np.abs((phi - iphi * dphi + np.pi) % (2.0 * np.pi) - np.pi)
        pv = np.maximum(0.0, 1.0 - dp / dphi)
        psi[k] = rv * pv
    return psi


_PSI = jnp.asarray(_disco_psi(_IN_SHAPE, _RADIUS, _KSHAPE))


def _conv_block(Sn, wa, wmp):
    n = _RB * _WP
    s1 = Sn[:, 1:_RB + 1, :]
    st3 = jnp.concatenate([Sn[:, 0:_RB, :], s1, Sn[:, 2:_RB + 2, :]],
                          axis=0).reshape(3 * _C, n)
    a1 = jnp.dot(wa, st3, preferred_element_type=jnp.float32)
    ap = jnp.dot(wmp, s1.reshape(_C, n), preferred_element_type=jnp.float32)
    a0 = ap[:_C]
    a2 = ap[_C:]
    z = jnp.zeros((_C, 1), jnp.float32)
    out = (a1
           + jnp.concatenate([z, a0[:, :-1]], axis=1)
           + jnp.concatenate([a2[:, 1:], z], axis=1))
    return out.reshape(_C, _RB, _WP)


def _body(x_ref, w1a_ref, w1mp_ref, w2a_ref, w2mp_ref, o_ref,
          xbuf, h1buf, stage, sem_in, sem_out):
    b = pl.program_id(0)
    cp = pltpu.make_async_copy(x_ref.at[b], xbuf, sem_in)
    cp.start()
    cp.wait()

    col = jax.lax.broadcasted_iota(jnp.int32, (1, 1, _WP), 2)
    valid = (col >= 1) & (col <= _W)

    w1a = w1a_ref[...]
    w1mp = w1mp_ref[...]
    w2a = w2a_ref[...]
    w2mp = w2mp_ref[...]

    zrow = jnp.zeros((_C, 1, _WP), jnp.bfloat16)
    h1buf[:, 0:1, :] = zrow
    h1buf[:, _H + 1:_H + 2, :] = zrow

    ssum = jnp.zeros((_C,), jnp.float32)
    ssq = jnp.zeros((_C,), jnp.float32)
    for blk in range(_NBLK):
        h0 = blk * _RB
        S = xbuf[:, h0:h0 + _RB + 2, :]
        o3 = _conv_block(S, w1a, w1mp)
        o3 = jnp.where(valid, o3, 0.0)
        h1buf[:, h0 + 1:h0 + _RB + 1, :] = o3.astype(jnp.bfloat16)
        ssum = ssum + jnp.sum(o3, axis=(1, 2))
        ssq = ssq + jnp.sum(o3 * o3, axis=(1, 2))
    mean1 = (ssum / _NPIX)[:, None, None]
    var1 = (ssq / _NPIX)[:, None, None] - mean1 * mean1
    rs1 = jax.lax.rsqrt(var1 + 1e-5)

    ssum = jnp.zeros((_C,), jnp.float32)
    ssq = jnp.zeros((_C,), jnp.float32)
    for blk in range(_NBLK):
        h0 = blk * _RB
        S = h1buf[:, h0:h0 + _RB + 2, :].astype(jnp.float32)
        Sn = (S - mean1) * rs1
        Sn = jnp.where(Sn >= 0, Sn, 0.2 * Sn)
        Sn = jnp.where(valid, Sn, 0.0)
        if blk == 0:
            Sn = jnp.concatenate([jnp.zeros((_C, 1, _WP), jnp.float32),
                                  Sn[:, 1:, :]], axis=1)
        if blk == _NBLK - 1:
            Sn = jnp.concatenate([Sn[:, :_RB + 1, :],
                                  jnp.zeros((_C, 1, _WP), jnp.float32)], axis=1)
        o3 = _conv_block(Sn.astype(jnp.bfloat16), w2a, w2mp)
        o3 = jnp.where(valid, o3, 0.0)
        xbuf[:, h0 + 1:h0 + _RB + 1, :] = o3.astype(jnp.bfloat16)
        ssum = ssum + jnp.sum(o3, axis=(1, 2))
        ssq = ssq + jnp.sum(o3 * o3, axis=(1, 2))
    mean2 = (ssum / _NPIX)[:, None, None]
    var2 = (ssq / _NPIX)[:, None, None] - mean2 * mean2
    rs2 = jax.lax.rsqrt(var2 + 1e-5)

    for blk in range(_NBLK):
        h0 = blk * _RB
        v = xbuf[:, h0 + 1:h0 + _RB + 1, :].astype(jnp.float32)
        vn = (v - mean2) * rs2
        vn = jnp.where(vn >= 0, vn, 0.2 * vn)
        stage[...] = vn[:, :, 1:_W + 1]
        cpo = pltpu.make_async_copy(
            stage, o_ref.at[b, :, h0:h0 + _RB, :], sem_out)
        cpo.start()
        cpo.wait()


def kernel(image, w1, w2):
    K1 = jnp.einsum("oik,kyx->oiyx", w1, _PSI)
    K2 = jnp.einsum("oik,kyx->oiyx", w2, _PSI)
    w1a = jnp.concatenate([K1[:, :, 0, 1], K1[:, :, 1, 1], K1[:, :, 2, 1]], axis=1)
    w2a = jnp.concatenate([K2[:, :, 0, 1], K2[:, :, 1, 1], K2[:, :, 2, 1]], axis=1)
    w1mp = jnp.concatenate([K1[:, :, 1, 0], K1[:, :, 1, 2]], axis=0)
    w2mp = jnp.concatenate([K2[:, :, 1, 0], K2[:, :, 1, 2]], axis=0)
    w1a, w2a, w1mp, w2mp = (w.astype(jnp.bfloat16)
                            for w in (w1a, w2a, w1mp, w2mp))

    xp = jnp.pad(image, ((0, 0), (0, 0), (1, 1), (1, _WP - _W - 1))
                 ).astype(jnp.bfloat16)

    hbm = pl.BlockSpec(memory_space=pltpu.MemorySpace.HBM)
    vmem = pl.BlockSpec(memory_space=pltpu.MemorySpace.VMEM)
    return pl.pallas_call(
        _body,
        grid=(2,),
        in_specs=[hbm, vmem, vmem, vmem, vmem],
        out_specs=hbm,
        out_shape=jax.ShapeDtypeStruct((2, _C, _H, _W), jnp.float32),
        scratch_shapes=[
            pltpu.VMEM((_C, _H + 2, _WP), jnp.bfloat16),
            pltpu.VMEM((_C, _H + 2, _WP), jnp.bfloat16),
            pltpu.VMEM((_C, _RB, _W), jnp.float32),
            pltpu.SemaphoreType.DMA,
            pltpu.SemaphoreType.DMA,
        ],
        compiler_params=pltpu.CompilerParams(
            dimension_semantics=("arbitrary",),
            vmem_limit_bytes=64 * 1024 * 1024,
        ),
    )(xp, w1a, w1mp, w2a, w2mp)


# speedup vs baseline: 5.1262x; 1.0013x over previous
import math

import jax
import jax.numpy as jnp
import numpy as np
from jax.experimental import pallas as pl
from jax.experimental.pallas import tpu as pltpu

_IN_SHAPE = (224, 224)
_RADIUS = 0.005
_KSHAPE = (3, 4)
_C = 96
_H, _W = _IN_SHAPE
_WP = 256
_RB = 16
_NBLK = _H // _RB
_NPIX = _H * _W


def _disco_psi(in_shape, radius_cutoff, kernel_shape):
    H, W = in_shape
    nr, nphi = kernel_shape
    n_basis = (nr - 1) * nphi + 1
    rh = max(1, int(math.floor(radius_cutoff * (H - 1))))
    rw = max(1, int(math.floor(radius_cutoff * (W - 1))))
    dy = np.arange(-rh, rh + 1) / float(H - 1)
    dx = np.arange(-rw, rw + 1) / float(W - 1)
    Y, X = np.meshgrid(dy, dx, indexing="ij")
    r = np.sqrt(X ** 2 + Y ** 2) / radius_cutoff
    phi = np.arctan2(Y, X) % (2.0 * np.pi)
    dr = 1.0 / (nr - 1)
    dphi = 2.0 * np.pi / nphi
    psi = np.zeros((n_basis, 2 * rh + 1, 2 * rw + 1), dtype=np.float32)
    support = (r <= 1.0).astype(np.float32)
    psi[0] = np.maximum(0.0, 1.0 - r / dr) * support
    for k in range(1, n_basis):
        ir = (k - 1) // nphi + 1
        iphi = (k - 1) % nphi
        rv = np.maximum(0.0, 1.0 - np.abs(r - ir * dr) / dr) * support
        dp = np.abs((phi - iphi * dphi + np.pi) % (2.0 * np.pi) - np.pi)
        pv = np.maximum(0.0, 1.0 - dp / dphi)
        psi[k] = rv * pv
    return psi


_PSI = jnp.asarray(_disco_psi(_IN_SHAPE, _RADIUS, _KSHAPE))


def _conv_block(Sn, wa, wmp):
    n = _RB * _WP
    s1 = Sn[:, 1:_RB + 1, :]
    st3 = jnp.concatenate([Sn[:, 0:_RB, :], s1, Sn[:, 2:_RB + 2, :]],
                          axis=0).reshape(3 * _C, n)
    a1 = jnp.dot(wa, st3, preferred_element_type=jnp.float32)
    ap = jnp.dot(wmp, s1.reshape(_C, n), preferred_element_type=jnp.float32)
    a0 = ap[:_C]
    a2 = ap[_C:]
    z = jnp.zeros((_C, 1), jnp.float32)
    out = (a1
           + jnp.concatenate([z, a0[:, :-1]], axis=1)
           + jnp.concatenate([a2[:, 1:], z], axis=1))
    return out.reshape(_C, _RB, _WP)


def _body(x_ref, w1a_ref, w1mp_ref, w2a_ref, w2mp_ref, o_ref,
          xbuf, h1buf, stage, sem_in, sem_out):
    b = pl.program_id(0)
    cp = pltpu.make_async_copy(x_ref.at[b], xbuf, sem_in)
    cp.start()
    cp.wait()

    col = jax.lax.broadcasted_iota(jnp.int32, (1, 1, _WP), 2)
    valid = (col >= 1) & (col <= _W)

    w1a = w1a_ref[...]
    w1mp = w1mp_ref[...]
    w2a = w2a_ref[...]
    w2mp = w2mp_ref[...]

    zrow = jnp.zeros((_C, 1, _WP), jnp.bfloat16)
    h1buf[:, 0:1, :] = zrow
    h1buf[:, _H + 1:_H + 2, :] = zrow

    ssum = jnp.zeros((_C,), jnp.float32)
    ssq = jnp.zeros((_C,), jnp.float32)
    for blk in range(_NBLK):
        h0 = blk * _RB
        S = xbuf[:, h0:h0 + _RB + 2, :]
        o3 = _conv_block(S, w1a, w1mp)
        o3 = jnp.where(valid, o3, 0.0)
        h1buf[:, h0 + 1:h0 + _RB + 1, :] = o3.astype(jnp.bfloat16)
        ssum = ssum + jnp.sum(o3, axis=(1, 2))
        ssq = ssq + jnp.sum(o3 * o3, axis=(1, 2))
    mean1 = (ssum / _NPIX)[:, None, None]
    var1 = (ssq / _NPIX)[:, None, None] - mean1 * mean1
    rs1 = jax.lax.rsqrt(var1 + 1e-5)

    ssum = jnp.zeros((_C,), jnp.float32)
    ssq = jnp.zeros((_C,), jnp.float32)
    for blk in range(_NBLK):
        h0 = blk * _RB
        S = h1buf[:, h0:h0 + _RB + 2, :].astype(jnp.float32)
        Sn = (S - mean1) * rs1
        Sn = jnp.where(Sn >= 0, Sn, 0.2 * Sn)
        Sn = jnp.where(valid, Sn, 0.0)
        if blk == 0:
            Sn = jnp.concatenate([jnp.zeros((_C, 1, _WP), jnp.float32),
                                  Sn[:, 1:, :]], axis=1)
        if blk == _NBLK - 1:
            Sn = jnp.concatenate([Sn[:, :_RB + 1, :],
                                  jnp.zeros((_C, 1, _WP), jnp.float32)], axis=1)
        o3 = _conv_block(Sn.astype(jnp.bfloat16), w2a, w2mp)
        o3 = jnp.where(valid, o3, 0.0)
        xbuf[:, h0 + 1:h0 + _RB + 1, :] = o3.astype(jnp.bfloat16)
        ssum = ssum + jnp.sum(o3, axis=(1, 2))
        ssq = ssq + jnp.sum(o3 * o3, axis=(1, 2))
    mean2 = (ssum / _NPIX)[:, None, None]
    var2 = (ssq / _NPIX)[:, None, None] - mean2 * mean2
    rs2 = jax.lax.rsqrt(var2 + 1e-5)

    for blk in range(_NBLK):
        h0 = blk * _RB
        v = xbuf[:, h0 + 1:h0 + _RB + 1, :].astype(jnp.float32)
        vn = (v - mean2) * rs2
        vn = jnp.where(vn >= 0, vn, 0.2 * vn)
        stage[...] = vn[:, :, 1:_W + 1]
        cpo = pltpu.make_async_copy(
            stage, o_ref.at[b, :, h0:h0 + _RB, :], sem_out)
        cpo.start()
        cpo.wait()


def kernel(image, w1, w2):
    K1 = jnp.einsum("oik,kyx->oiyx", w1, _PSI)
    K2 = jnp.einsum("oik,kyx->oiyx", w2, _PSI)
    w1a = jnp.concatenate([K1[:, :, 0, 1], K1[:, :, 1, 1], K1[:, :, 2, 1]], axis=1)
    w2a = jnp.concatenate([K2[:, :, 0, 1], K2[:, :, 1, 1], K2[:, :, 2, 1]], axis=1)
    w1mp = jnp.concatenate([K1[:, :, 1, 0], K1[:, :, 1, 2]], axis=0)
    w2mp = jnp.concatenate([K2[:, :, 1, 0], K2[:, :, 1, 2]], axis=0)
    w1a, w2a, w1mp, w2mp = (w.astype(jnp.bfloat16)
                            for w in (w1a, w2a, w1mp, w2mp))

    xp = jnp.pad(image, ((0, 0), (0, 0), (1, 1), (1, _WP - _W - 1))
                 ).astype(jnp.bfloat16)

    hbm = pl.BlockSpec(memory_space=pltpu.MemorySpace.HBM)
    vmem = pl.BlockSpec(memory_space=pltpu.MemorySpace.VMEM)
    return pl.pallas_call(
        _body,
        grid=(2,),
        in_specs=[hbm, vmem, vmem, vmem, vmem],
        out_specs=hbm,
        out_shape=jax.ShapeDtypeStruct((2, _C, _H, _W), jnp.float32),
        scratch_shapes=[
            pltpu.VMEM((_C, _H + 2, _WP), jnp.bfloat16),
            pltpu.VMEM((_C, _H + 2, _WP), jnp.bfloat16),
            pltpu.VMEM((_C, _RB, _W), jnp.float32),
            pltpu.SemaphoreType.DMA,
            pltpu.SemaphoreType.DMA,
        ],
        compiler_params=pltpu.CompilerParams(
            dimension_semantics=("parallel",),
            vmem_limit_bytes=64 * 1024 * 1024,
        ),
    )(xp, w1a, w1mp, w2a, w2mp)
